# Initial kernel scaffold; baseline (speedup 1.0000x reference)
import jax
import jax.numpy as jnp
from jax.experimental import pallas as pl


def kernel(predicted_probabilities, weights_lukasiewicz):
    raise NotImplementedError("write your pallas kernel here")


# speedup vs baseline: 16.2303x; 16.2303x over previous
import jax
import jax.numpy as jnp
from jax import lax
from jax.experimental import pallas as pl
from jax.experimental.pallas import tpu as pltpu
from jax.experimental.pallas import tpu_sc as plsc

NUM_TYPES = 1365
BATCH = 4096
NUM_PARENTS = 341
W_PAD = 1376

_NC = 2
_NS = 16
_NW = _NC * _NS
ROWS_PER_PASS = 64
PASSES = BATCH // (_NW * ROWS_PER_PASS)


def _tree_body(x_hbm, w_hbm, out_hbm, w_v, sw_v, tile_v):
    wid = lax.axis_index("s") * _NC + lax.axis_index("c")

    pltpu.sync_copy(w_hbm, w_v)
    for i in range(W_PAD // 16):
        wv = w_v[pl.ds(i * 16, 16)]
        sw_v[pl.ds(i * 16, 16)] = 1.0 / (1.0 + jnp.exp(-wv))

    iota = lax.iota(jnp.int32, 16)
    zeros16 = jnp.zeros((16,), jnp.int32)

    for pass_ in range(PASSES):
        row0 = (wid * PASSES + pass_) * ROWS_PER_PASS
        pltpu.sync_copy(x_hbm.at[pl.ds(row0, ROWS_PER_PASS)], tile_v)

        def parent_step(i, carry):
            p = NUM_PARENTS - 1 - i
            base = 4 * p + 1
            ws = [plsc.load_gather(sw_v, [zeros16 + (base + k)])
                  for k in range(4)]
            pidx = zeros16 + p
            for c in range(ROWS_PER_PASS // 16):
                rows = iota + (c * 16)
                acc = jnp.zeros((16,), jnp.float32)
                for k in range(4):
                    v = plsc.load_gather(tile_v, [rows, zeros16 + (base + k)])
                    acc = acc + ws[k] * v
                acc = jnp.clip(acc, 0.0, 1.0)
                plsc.store_scatter(tile_v, [rows, pidx], acc)
            return carry

        lax.fori_loop(0, NUM_PARENTS, parent_step, 0)
        pltpu.sync_copy(tile_v, out_hbm.at[pl.ds(row0, ROWS_PER_PASS)])


def kernel(predicted_probabilities, weights_lukasiewicz):
    w_pad = jnp.zeros((W_PAD,), jnp.float32)
    w_pad = w_pad.at[:NUM_TYPES].set(weights_lukasiewicz.astype(jnp.float32))

    mesh = plsc.VectorSubcoreMesh(core_axis_name="c", subcore_axis_name="s")
    f = pl.kernel(
        _tree_body,
        out_type=jax.ShapeDtypeStruct((BATCH, NUM_TYPES), jnp.float32),
        mesh=mesh,
        compiler_params=pltpu.CompilerParams(needs_layout_passes=False),
        scratch_types=[
            pltpu.VMEM((W_PAD,), jnp.float32),
            pltpu.VMEM((W_PAD,), jnp.float32),
            pltpu.VMEM((ROWS_PER_PASS, NUM_TYPES), jnp.float32),
        ],
    )
    return f(predicted_probabilities, w_pad)
